# baseline (speedup 1.0000x reference)
import jax
import jax.numpy as jnp
from jax.experimental import pallas as pl

_P = 0.1


def _body(rand_ref, mask_ref, out_ref):
    keep = (rand_ref[...] >= _P).astype(jnp.float32)
    out_ref[...] = mask_ref[...] * keep


def kernel(x, mask, rand_t):
    C, T, V = mask.shape
    rand2d = rand_t.reshape(T, 1)
    out = pl.pallas_call(
        _body,
        grid=(C,),
        in_specs=[
            pl.BlockSpec((T, 1), lambda i: (0, 0)),
            pl.BlockSpec((1, T, V), lambda i: (i, 0, 0)),
        ],
        out_specs=pl.BlockSpec((1, T, V), lambda i: (i, 0, 0)),
        out_shape=jax.ShapeDtypeStruct((C, T, V), jnp.float32),
    )(rand2d, mask)
    return (x, out)


# speedup vs baseline: 1.1319x; 1.1319x over previous
import jax
import jax.numpy as jnp
from jax.experimental import pallas as pl

_P = 0.1


def _expand_body(rand_ref, keep_ref):
    keep = (rand_ref[...] >= _P).astype(jnp.float32)
    keep_ref[...] = jnp.broadcast_to(keep, keep_ref.shape)


def _mul_body(keep_ref, mask_ref, out_ref):
    out_ref[...] = mask_ref[...] * keep_ref[...][None]


def kernel(x, mask, rand_t):
    C, T, V = mask.shape
    N = T * V
    ROWS, COLS = 8, N // 8
    CB = 8

    keep_tv = pl.pallas_call(
        _expand_body,
        out_shape=jax.ShapeDtypeStruct((T, V), jnp.float32),
    )(rand_t.reshape(T, 1))

    keep2d = keep_tv.reshape(ROWS, COLS)
    mask3d = mask.reshape(C, ROWS, COLS)

    out = pl.pallas_call(
        _mul_body,
        grid=(C // CB,),
        in_specs=[
            pl.BlockSpec((ROWS, COLS), lambda i: (0, 0)),
            pl.BlockSpec((CB, ROWS, COLS), lambda i: (i, 0, 0)),
        ],
        out_specs=pl.BlockSpec((CB, ROWS, COLS), lambda i: (i, 0, 0)),
        out_shape=jax.ShapeDtypeStruct((C, ROWS, COLS), jnp.float32),
    )(keep2d, mask3d)
    return (x, out.reshape(C, T, V))


# speedup vs baseline: 1.1502x; 1.0162x over previous
import jax
import jax.numpy as jnp
from jax.experimental import pallas as pl

_P = 0.1


def _expand_body(rand_ref, keep_ref):
    keep = (rand_ref[...] >= _P).astype(jnp.float32)
    keep_ref[...] = jnp.broadcast_to(keep, keep_ref.shape)


def _mul_body(keep_ref, mask_ref, out_ref):
    out_ref[...] = mask_ref[...] * keep_ref[...][None]


def kernel(x, mask, rand_t):
    C, T, V = mask.shape
    CB = 4

    keep_tv = pl.pallas_call(
        _expand_body,
        out_shape=jax.ShapeDtypeStruct((T, V), jnp.float32),
    )(rand_t.reshape(T, 1))

    out = pl.pallas_call(
        _mul_body,
        grid=(C // CB,),
        in_specs=[
            pl.BlockSpec((T, V), lambda i: (0, 0)),
            pl.BlockSpec((CB, T, V), lambda i: (i, 0, 0)),
        ],
        out_specs=pl.BlockSpec((CB, T, V), lambda i: (i, 0, 0)),
        out_shape=jax.ShapeDtypeStruct((C, T, V), jnp.float32),
    )(keep_tv, mask)
    return (x, out)
